# baseline (speedup 1.0000x reference)
import numpy as np

import jax
import jax.numpy as jnp
from jax.experimental import pallas as pl
from jax.experimental.pallas import tpu as pltpu
from jax.experimental.pallas import tpu_sc as plsc


def _argmin_body(x_ref, emb_ref, idx_ref):
    xb = x_ref[...]
    e = emb_ref[...]
    s = jax.lax.dot_general(
        xb, e, (((1,), (1,)), ((), ())),
        preferred_element_type=jnp.float32,
        precision=jax.lax.Precision.DEFAULT)
    z2 = jnp.sum(xb * xb, axis=1, keepdims=True)
    e2 = jnp.sum(e * e, axis=1)[None, :]
    dist = (z2 + e2) - 2.0 * s
    m = jnp.min(dist, axis=1, keepdims=True)
    k = dist.shape[1]
    iota = jax.lax.broadcasted_iota(jnp.int32, dist.shape, 1)
    idx = jnp.min(jnp.where(dist == m, iota, k), axis=1)
    idx_ref[0, 0, :] = idx.astype(jnp.int32)


def _nearest_indices(x, emb):
    n, d = x.shape
    k = emb.shape[0]
    tb = 256
    nb = n // tb
    idx3 = pl.pallas_call(
        _argmin_body,
        grid=(nb,),
        in_specs=[
            pl.BlockSpec((tb, d), lambda i: (i, 0)),
            pl.BlockSpec((k, d), lambda i: (0, 0)),
        ],
        out_specs=pl.BlockSpec((1, 1, tb), lambda i: (i, 0, 0)),
        out_shape=jax.ShapeDtypeStruct((nb, 1, tb), jnp.int32),
    )(x, emb)
    return idx3.reshape(n)


def _sc_gather(emb, idx):
    n = idx.shape[0]
    d = emb.shape[1]
    window = 128
    mesh = plsc.VectorSubcoreMesh(core_axis_name="core",
                                  subcore_axis_name="subcore")
    idx2 = idx.reshape(1, n)

    @pl.kernel(out_type=jax.ShapeDtypeStruct((n, d), emb.dtype), mesh=mesh)
    def gather_kernel(emb_hbm, i_hbm, o_hbm):
        def body(i_vmem, o_vmem):
            pltpu.sync_copy(emb_hbm.at[i_vmem.at[0]], o_vmem)

        pltpu.emit_pipeline(
            body,
            grid=(n // window,),
            in_specs=[pl.BlockSpec((1, window), index_map=lambda i: (0, i))],
            out_specs=[pl.BlockSpec((window, d), index_map=lambda i: (i, 0))],
            core_axis_name=("core", "subcore"),
            dimension_semantics=(pltpu.PARALLEL,),
        )(i_hbm, o_hbm)

    return gather_kernel(emb, idx2)


def _expand_body(wt_ref, m_ref, *, cin, cout, hin):
    hout = 2 * hin
    m_ref[...] = jnp.zeros(m_ref.shape, m_ref.dtype)
    for kh in range(4):
        for kw in range(4):
            w = wt_ref[(kh * 4 + kw) * cin:(kh * 4 + kw + 1) * cin, :]
            for ih in range(hin):
                oh = 2 * ih - 1 + kh
                if not 0 <= oh < hout:
                    continue
                for iw in range(hin):
                    ow = 2 * iw - 1 + kw
                    if not 0 <= ow < hout:
                        continue
                    r = (ih * hin + iw) * cin
                    c = (oh * hout + ow) * cout
                    m_ref[r:r + cin, c:c + cout] = w


def _expand_deconv(w_bf16, hin):
    import functools
    cin, cout = w_bf16.shape[0], w_bf16.shape[1]
    hout = 2 * hin
    wt = jnp.transpose(w_bf16, (2, 3, 0, 1)).reshape(16 * cin, cout)
    return pl.pallas_call(
        functools.partial(_expand_body, cin=cin, cout=cout, hin=hin),
        out_shape=jax.ShapeDtypeStruct((hin * hin * cin, hout * hout * cout),
                                       w_bf16.dtype),
    )(wt)


def _placement_const(hin):
    hout = 2 * hin
    g = np.zeros((hin * hin, 16, hout * hout), np.float32)
    for ih in range(hin):
        for iw in range(hin):
            for kh in range(4):
                for kw in range(4):
                    oh, ow = 2 * ih - 1 + kh, 2 * iw - 1 + kw
                    if 0 <= oh < hout and 0 <= ow < hout:
                        g[ih * hin + iw, kh * 4 + kw, oh * hout + ow] = 1.0
    return jnp.asarray(g.astype(jnp.bfloat16))


def _expand_last(w_bf16, hin):
    cin, cout = w_bf16.shape[0], w_bf16.shape[1]
    hout = 2 * hin
    wt = jnp.transpose(w_bf16, (2, 3, 0, 1)).reshape(16, cin * cout)
    s = hin * hin
    wt_b = jnp.broadcast_to(wt[None], (s, 16, cin * cout))
    g = _placement_const(hin)
    m = jax.lax.dot_general(
        wt_b, g, (((1,), (1,)), ((0,), (0,))),
        preferred_element_type=jnp.float32)
    m = m.reshape(s, cin, cout, hout * hout).astype(w_bf16.dtype)
    return m.reshape(s * cin, cout * hout * hout)


def _decoder_body(zq_ref, m1_ref, m2_ref, m3_ref, m4_ref,
                  b1_ref, b2_ref, b3_ref, b4_ref, out_ref):
    f32 = jnp.float32
    bf16 = jnp.bfloat16
    h = zq_ref[...].astype(bf16)
    h = jnp.dot(h, m1_ref[...], preferred_element_type=f32) + b1_ref[...]
    h = jnp.maximum(h, 0.0).astype(bf16)
    h = jnp.dot(h, m2_ref[...], preferred_element_type=f32) + b2_ref[...]
    h = jnp.maximum(h, 0.0).astype(bf16)
    h = jnp.dot(h, m3_ref[...], preferred_element_type=f32) + b3_ref[...]
    h = jnp.maximum(h, 0.0).astype(bf16)
    h = jnp.dot(h, m4_ref[...], preferred_element_type=f32) + b4_ref[...]
    out_ref[...] = jax.nn.sigmoid(h)


def _decoder(zq, W1, b1, W2, b2, W3, b3, W4, b4):
    n = zq.shape[0]
    bf16 = jnp.bfloat16
    w1b, w2b = W1.astype(bf16), W2.astype(bf16)
    w3b, w4b = W3.astype(bf16), W4.astype(bf16)
    m1 = jnp.concatenate([w1b[:, :, 1, 1], w1b[:, :, 1, 2],
                          w1b[:, :, 2, 1], w1b[:, :, 2, 2]], axis=1)
    m2 = _expand_deconv(w2b, 2)
    m3 = _expand_deconv(w3b, 4)
    m4 = _expand_last(w4b, 8)
    b1f = jnp.tile(b1, 4).reshape(1, -1)
    b2f = jnp.tile(b2, 16).reshape(1, -1)
    b3f = jnp.tile(b3, 64).reshape(1, -1)
    b4f = jnp.repeat(b4, 256).reshape(1, -1)

    tb = 256
    nb = n // tb
    full = lambda a: pl.BlockSpec(a.shape, lambda i: tuple(0 for _ in a.shape))
    out = pl.pallas_call(
        _decoder_body,
        grid=(nb,),
        in_specs=[pl.BlockSpec((tb, zq.shape[1]), lambda i: (i, 0)),
                  full(m1), full(m2), full(m3), full(m4),
                  full(b1f), full(b2f), full(b3f), full(b4f)],
        out_specs=pl.BlockSpec((tb, m4.shape[1]), lambda i: (i, 0)),
        out_shape=jax.ShapeDtypeStruct((n, m4.shape[1]), jnp.float32),
    )(zq, m1, m2, m3, m4, b1f, b2f, b3f, b4f)
    return out.reshape(n, W4.shape[1], 16, 16)


def kernel(x, emb, W1, b1, W2, b2, W3, b3, W4, b4):
    n, d = x.shape
    idx = _nearest_indices(x, emb)
    zq = _sc_gather(emb, idx)
    x_recon = _decoder(zq, W1, b1, W2, b2, W3, b3, W4, b4)
    z = x.reshape(n, d, 1, 1)
    return (x_recon, z, zq.reshape(n, d, 1, 1), idx)


# speedup vs baseline: 1.1140x; 1.1140x over previous
import functools

import numpy as np

import jax
import jax.numpy as jnp
from jax.experimental import pallas as pl
from jax.experimental.pallas import tpu as pltpu
from jax.experimental.pallas import tpu_sc as plsc


def _argmin_body(x_ref, emb_ref, idx_ref, e2_ref):
    @pl.when(pl.program_id(0) == 0)
    def _():
        e = emb_ref[...]
        e2_ref[...] = jnp.sum(e * e, axis=1)[None, :]

    xb = x_ref[...]
    s = jax.lax.dot_general(
        xb, emb_ref[...], (((1,), (1,)), ((), ())),
        preferred_element_type=jnp.float32,
        precision=jax.lax.Precision.DEFAULT)
    z2 = jnp.sum(xb * xb, axis=1, keepdims=True)
    dist = (z2 + e2_ref[...]) - 2.0 * s
    m = jnp.min(dist, axis=1, keepdims=True)
    k = dist.shape[1]
    iota = jax.lax.broadcasted_iota(jnp.int32, dist.shape, 1)
    idx = jnp.min(jnp.where(dist == m, iota, k), axis=1)
    idx_ref[0, 0, :] = idx.astype(jnp.int32)


def _nearest_indices(x, emb):
    n, d = x.shape
    k = emb.shape[0]
    tb = 256
    nb = n // tb
    idx3 = pl.pallas_call(
        _argmin_body,
        grid=(nb,),
        in_specs=[
            pl.BlockSpec((tb, d), lambda i: (i, 0)),
            pl.BlockSpec((k, d), lambda i: (0, 0)),
        ],
        out_specs=pl.BlockSpec((1, 1, tb), lambda i: (i, 0, 0)),
        out_shape=jax.ShapeDtypeStruct((nb, 1, tb), jnp.int32),
        scratch_shapes=[pltpu.VMEM((1, k), jnp.float32)],
    )(x, emb)
    return idx3.reshape(n)


def _sc_gather(emb, idx):
    n = idx.shape[0]
    d = emb.shape[1]
    window = 128
    mesh = plsc.VectorSubcoreMesh(core_axis_name="core",
                                  subcore_axis_name="subcore")
    idx2 = idx.reshape(1, n)

    @pl.kernel(out_type=jax.ShapeDtypeStruct((n, d), emb.dtype), mesh=mesh)
    def gather_kernel(emb_hbm, i_hbm, o_hbm):
        def body(i_vmem, o_vmem):
            pltpu.sync_copy(emb_hbm.at[i_vmem.at[0]], o_vmem)

        pltpu.emit_pipeline(
            body,
            grid=(n // window,),
            in_specs=[pl.BlockSpec((1, window), index_map=lambda i: (0, i))],
            out_specs=[pl.BlockSpec((window, d), index_map=lambda i: (i, 0))],
            core_axis_name=("core", "subcore"),
            dimension_semantics=(pltpu.PARALLEL,),
        )(i_hbm, o_hbm)

    return gather_kernel(emb, idx2)


def _expand_stores(wt_ref, m_ref, cin, cout, hin):
    hout = 2 * hin
    m_ref[...] = jnp.zeros(m_ref.shape, m_ref.dtype)
    for kh in range(4):
        for kw in range(4):
            w = wt_ref[(kh * 4 + kw) * cin:(kh * 4 + kw + 1) * cin, :]
            for ih in range(hin):
                oh = 2 * ih - 1 + kh
                if not 0 <= oh < hout:
                    continue
                for iw in range(hin):
                    ow = 2 * iw - 1 + kw
                    if not 0 <= ow < hout:
                        continue
                    r = (ih * hin + iw) * cin
                    c = (oh * hout + ow) * cout
                    m_ref[r:r + cin, c:c + cout] = w


def _placement_const(hin):
    hout = 2 * hin
    g = np.zeros((hin * hin, 16, hout * hout), np.float32)
    for ih in range(hin):
        for iw in range(hin):
            for kh in range(4):
                for kw in range(4):
                    oh, ow = 2 * ih - 1 + kh, 2 * iw - 1 + kw
                    if 0 <= oh < hout and 0 <= ow < hout:
                        g[ih * hin + iw, kh * 4 + kw, oh * hout + ow] = 1.0
    return jnp.asarray(g.astype(jnp.bfloat16))


def _expand_last(w_bf16, hin):
    cin, cout = w_bf16.shape[0], w_bf16.shape[1]
    hout = 2 * hin
    wt = jnp.transpose(w_bf16, (2, 3, 0, 1)).reshape(16, cin * cout)
    s = hin * hin
    wt_b = jnp.broadcast_to(wt[None], (s, 16, cin * cout))
    g = _placement_const(hin)
    m = jax.lax.dot_general(
        wt_b, g, (((1,), (1,)), ((0,), (0,))),
        preferred_element_type=jnp.float32)
    m = m.reshape(s, cin, cout, hout * hout).astype(w_bf16.dtype)
    return m.reshape(s * cin, cout * hout * hout)


def _decoder_body(zq_ref, wt1_ref, wt2_ref, wt3_ref, m4_ref,
                  b1_ref, b2_ref, b3_ref, b4_ref, out_ref,
                  m1_s, m2_s, m3_s, b1_s, b2_s, b3_s):
    f32 = jnp.float32
    bf16 = jnp.bfloat16

    @pl.when(pl.program_id(0) == 0)
    def _():
        for p in range(4):
            m1_s[:, p * 256:(p + 1) * 256] = wt1_ref[p * 256:(p + 1) * 256, :]
            b1_s[:, p * 256:(p + 1) * 256] = b1_ref[...]
        _expand_stores(wt2_ref, m2_s, 256, 128, 2)
        _expand_stores(wt3_ref, m3_s, 128, 64, 4)
        for p in range(16):
            b2_s[:, p * 128:(p + 1) * 128] = b2_ref[...]
        for p in range(64):
            b3_s[:, p * 64:(p + 1) * 64] = b3_ref[...]

    h = zq_ref[...].astype(bf16)
    h = jnp.dot(h, m1_s[...], preferred_element_type=f32) + b1_s[...]
    h = jnp.maximum(h, 0.0).astype(bf16)
    h = jnp.dot(h, m2_s[...], preferred_element_type=f32) + b2_s[...]
    h = jnp.maximum(h, 0.0).astype(bf16)
    h = jnp.dot(h, m3_s[...], preferred_element_type=f32) + b3_s[...]
    h = jnp.maximum(h, 0.0).astype(bf16)
    h = jnp.dot(h, m4_ref[...], preferred_element_type=f32) + b4_ref[...]
    out_ref[...] = jax.nn.sigmoid(h)


def _decoder(zq, W1, b1, W2, b2, W3, b3, W4, b4):
    n = zq.shape[0]
    bf16 = jnp.bfloat16
    wt1 = jnp.transpose(W1.astype(bf16)[:, :, 1:3, 1:3],
                        (2, 3, 0, 1)).reshape(4 * 256, 256)
    wt2 = jnp.transpose(W2.astype(bf16), (2, 3, 0, 1)).reshape(16 * 256, 128)
    wt3 = jnp.transpose(W3.astype(bf16), (2, 3, 0, 1)).reshape(16 * 128, 64)
    m4 = _expand_last(W4.astype(bf16), 8)
    b1r = b1.reshape(1, -1)
    b2r = b2.reshape(1, -1)
    b3r = b3.reshape(1, -1)
    b4f = jnp.repeat(b4, 256).reshape(1, -1)

    tb = 256
    nb = n // tb
    full = lambda a: pl.BlockSpec(a.shape, lambda i: tuple(0 for _ in a.shape))
    out = pl.pallas_call(
        _decoder_body,
        grid=(nb,),
        in_specs=[pl.BlockSpec((tb, zq.shape[1]), lambda i: (i, 0)),
                  full(wt1), full(wt2), full(wt3), full(m4),
                  full(b1r), full(b2r), full(b3r), full(b4f)],
        out_specs=pl.BlockSpec((tb, m4.shape[1]), lambda i: (i, 0)),
        out_shape=jax.ShapeDtypeStruct((n, m4.shape[1]), jnp.float32),
        scratch_shapes=[pltpu.VMEM((256, 1024), bf16),
                        pltpu.VMEM((1024, 2048), bf16),
                        pltpu.VMEM((2048, 4096), bf16),
                        pltpu.VMEM((1, 1024), jnp.float32),
                        pltpu.VMEM((1, 2048), jnp.float32),
                        pltpu.VMEM((1, 4096), jnp.float32)],
    )(zq, wt1, wt2, wt3, m4, b1r, b2r, b3r, b4f)
    return out.reshape(n, W4.shape[1], 16, 16)


def kernel(x, emb, W1, b1, W2, b2, W3, b3, W4, b4):
    n, d = x.shape
    idx = _nearest_indices(x, emb)
    zq = _sc_gather(emb, idx)
    x_recon = _decoder(zq, W1, b1, W2, b2, W3, b3, W4, b4)
    z = x.reshape(n, d, 1, 1)
    return (x_recon, z, zq.reshape(n, d, 1, 1), idx)


# speedup vs baseline: 1.1344x; 1.0183x over previous
import functools

import numpy as np

import jax
import jax.numpy as jnp
from jax.experimental import pallas as pl
from jax.experimental.pallas import tpu as pltpu
from jax.experimental.pallas import tpu_sc as plsc


def _argmin_body(x_ref, emb_ref, idx_ref, e2_ref):
    @pl.when(pl.program_id(0) == 0)
    def _():
        e = emb_ref[...]
        e2_ref[...] = jnp.sum(e * e, axis=1)[None, :]

    xb = x_ref[...]
    tb = xb.shape[0]
    k = emb_ref.shape[0]
    z2 = jnp.sum(xb * xb, axis=1, keepdims=True)

    ck = 1024
    run_min = None
    for c in range(k // ck):
        s = jax.lax.dot_general(
            xb, emb_ref[c * ck:(c + 1) * ck, :], (((1,), (1,)), ((), ())),
            preferred_element_type=jnp.float32,
            precision=jax.lax.Precision.DEFAULT)
        dist = (z2 + e2_ref[0:1, c * ck:(c + 1) * ck]) - 2.0 * s
        ii = jax.lax.broadcasted_iota(jnp.int32, (tb, ck), 1) + c * ck
        if run_min is None:
            run_min, run_idx = dist, ii
        else:
            better = dist < run_min
            run_min = jnp.where(better, dist, run_min)
            run_idx = jnp.where(better, ii, run_idx)
    m = jnp.min(run_min, axis=1, keepdims=True)
    idx = jnp.min(jnp.where(run_min == m, run_idx, k), axis=1)
    idx_ref[0, 0, :] = idx.astype(jnp.int32)


def _nearest_indices(x, emb):
    n, d = x.shape
    k = emb.shape[0]
    tb = 256
    nb = n // tb
    idx3 = pl.pallas_call(
        _argmin_body,
        grid=(nb,),
        in_specs=[
            pl.BlockSpec((tb, d), lambda i: (i, 0)),
            pl.BlockSpec((k, d), lambda i: (0, 0)),
        ],
        out_specs=pl.BlockSpec((1, 1, tb), lambda i: (i, 0, 0)),
        out_shape=jax.ShapeDtypeStruct((nb, 1, tb), jnp.int32),
        scratch_shapes=[pltpu.VMEM((1, k), jnp.float32)],
    )(x, emb)
    return idx3.reshape(n)


def _sc_gather(emb, idx):
    n = idx.shape[0]
    d = emb.shape[1]
    window = 128
    mesh = plsc.VectorSubcoreMesh(core_axis_name="core",
                                  subcore_axis_name="subcore")
    idx2 = idx.reshape(1, n)

    @pl.kernel(out_type=jax.ShapeDtypeStruct((n, d), emb.dtype), mesh=mesh)
    def gather_kernel(emb_hbm, i_hbm, o_hbm):
        def body(i_vmem, o_vmem):
            pltpu.sync_copy(emb_hbm.at[i_vmem.at[0]], o_vmem)

        pltpu.emit_pipeline(
            body,
            grid=(n // window,),
            in_specs=[pl.BlockSpec((1, window), index_map=lambda i: (0, i))],
            out_specs=[pl.BlockSpec((window, d), index_map=lambda i: (i, 0))],
            core_axis_name=("core", "subcore"),
            dimension_semantics=(pltpu.PARALLEL,),
        )(i_hbm, o_hbm)

    return gather_kernel(emb, idx2)


def _expand_stores(wt_ref, m_ref, cin, cout, hin):
    hout = 2 * hin
    m_ref[...] = jnp.zeros(m_ref.shape, m_ref.dtype)
    for kh in range(4):
        for kw in range(4):
            w = wt_ref[(kh * 4 + kw) * cin:(kh * 4 + kw + 1) * cin, :]
            for ih in range(hin):
                oh = 2 * ih - 1 + kh
                if not 0 <= oh < hout:
                    continue
                for iw in range(hin):
                    ow = 2 * iw - 1 + kw
                    if not 0 <= ow < hout:
                        continue
                    r = (ih * hin + iw) * cin
                    c = (oh * hout + ow) * cout
                    m_ref[r:r + cin, c:c + cout] = w


def _placement_q():
    hin, hout, cout = 8, 16, 3
    q = np.zeros((hin * hin, cout * 16, cout * hout * hout), np.float32)
    for ih in range(hin):
        for iw in range(hin):
            s = ih * hin + iw
            for kh in range(4):
                for kw in range(4):
                    oh, ow = 2 * ih - 1 + kh, 2 * iw - 1 + kw
                    if 0 <= oh < hout and 0 <= ow < hout:
                        k = kh * 4 + kw
                        p = oh * hout + ow
                        for o in range(cout):
                            q[s, o * 16 + k, o * hout * hout + p] = 1.0
    return jnp.asarray(q.astype(jnp.bfloat16))


def _decoder_body(zq_ref, wt1_ref, wt2_ref, wt3_ref, w4_ref, q_ref,
                  b1_ref, b2_ref, b3_ref, b4_ref, out_ref,
                  m1_s, m2_s, m3_s, m4_s, b1_s, b2_s, b3_s, b4_s):
    f32 = jnp.float32
    bf16 = jnp.bfloat16

    @pl.when(pl.program_id(0) == 0)
    def _():
        for p in range(4):
            m1_s[:, p * 256:(p + 1) * 256] = wt1_ref[p * 256:(p + 1) * 256, :]
            b1_s[:, p * 256:(p + 1) * 256] = b1_ref[...]
        _expand_stores(wt2_ref, m2_s, 256, 128, 2)
        _expand_stores(wt3_ref, m3_s, 128, 64, 4)
        for p in range(16):
            b2_s[:, p * 128:(p + 1) * 128] = b2_ref[...]
        for p in range(64):
            b3_s[:, p * 64:(p + 1) * 64] = b3_ref[...]
        w4 = w4_ref[...]
        for s in range(64):
            m4_s[s * 64:(s + 1) * 64, :] = jnp.dot(
                w4, q_ref[s], preferred_element_type=f32).astype(bf16)
        for o in range(3):
            b4_s[:, o * 256:(o + 1) * 256] = jnp.broadcast_to(
                b4_ref[0:1, o:o + 1], (1, 256))

    h = zq_ref[...].astype(bf16)
    h = jnp.dot(h, m1_s[...], preferred_element_type=f32) + b1_s[...]
    h = jnp.maximum(h, 0.0).astype(bf16)
    h = jnp.dot(h, m2_s[...], preferred_element_type=f32) + b2_s[...]
    h = jnp.maximum(h, 0.0).astype(bf16)
    h = jnp.dot(h, m3_s[...], preferred_element_type=f32) + b3_s[...]
    h = jnp.maximum(h, 0.0).astype(bf16)
    h = jnp.dot(h, m4_s[...], preferred_element_type=f32) + b4_s[...]
    out_ref[...] = jax.nn.sigmoid(h)


def _decoder(zq, W1, b1, W2, b2, W3, b3, W4, b4):
    n = zq.shape[0]
    bf16 = jnp.bfloat16
    wt1 = jnp.transpose(W1.astype(bf16)[:, :, 1:3, 1:3],
                        (2, 3, 0, 1)).reshape(4 * 256, 256)
    wt2 = jnp.transpose(W2.astype(bf16), (2, 3, 0, 1)).reshape(16 * 256, 128)
    wt3 = jnp.transpose(W3.astype(bf16), (2, 3, 0, 1)).reshape(16 * 128, 64)
    w4r = W4.astype(bf16).reshape(64, 48)
    q = _placement_q()
    b1r = b1.reshape(1, -1)
    b2r = b2.reshape(1, -1)
    b3r = b3.reshape(1, -1)
    b4r = b4.reshape(1, -1)

    tb = 256
    nb = n // tb
    full = lambda a: pl.BlockSpec(a.shape, lambda i: tuple(0 for _ in a.shape))
    out = pl.pallas_call(
        _decoder_body,
        grid=(nb,),
        in_specs=[pl.BlockSpec((tb, zq.shape[1]), lambda i: (i, 0)),
                  full(wt1), full(wt2), full(wt3), full(w4r), full(q),
                  full(b1r), full(b2r), full(b3r), full(b4r)],
        out_specs=pl.BlockSpec((tb, 768), lambda i: (i, 0)),
        out_shape=jax.ShapeDtypeStruct((n, 768), jnp.float32),
        scratch_shapes=[pltpu.VMEM((256, 1024), bf16),
                        pltpu.VMEM((1024, 2048), bf16),
                        pltpu.VMEM((2048, 4096), bf16),
                        pltpu.VMEM((4096, 768), bf16),
                        pltpu.VMEM((1, 1024), jnp.float32),
                        pltpu.VMEM((1, 2048), jnp.float32),
                        pltpu.VMEM((1, 4096), jnp.float32),
                        pltpu.VMEM((1, 768), jnp.float32)],
    )(zq, wt1, wt2, wt3, w4r, q, b1r, b2r, b3r, b4r)
    return out.reshape(n, W4.shape[1], 16, 16)


def kernel(x, emb, W1, b1, W2, b2, W3, b3, W4, b4):
    n, d = x.shape
    idx = _nearest_indices(x, emb)
    zq = _sc_gather(emb, idx)
    x_recon = _decoder(zq, W1, b1, W2, b2, W3, b3, W4, b4)
    z = x.reshape(n, d, 1, 1)
    return (x_recon, z, zq.reshape(n, d, 1, 1), idx)


# speedup vs baseline: 1.1848x; 1.0444x over previous
import functools

import numpy as np

import jax
import jax.numpy as jnp
from jax.experimental import pallas as pl
from jax.experimental.pallas import tpu as pltpu
from jax.experimental.pallas import tpu_sc as plsc


def _argmin_body(x_ref, emb_ref, idx_ref, e2_ref):
    @pl.when(pl.program_id(0) == 0)
    def _():
        e = emb_ref[...]
        e2_ref[...] = jnp.sum(e * e, axis=1)[None, :]

    xb = x_ref[...]
    tb = xb.shape[0]
    k = emb_ref.shape[0]
    z2 = jnp.sum(xb * xb, axis=1, keepdims=True)

    ck = 1024
    lanes = 128
    base_iota = jax.lax.broadcasted_iota(jnp.int32, (tb, lanes), 1)
    gmin = gidx = None
    for c in range(k // ck):
        s = jax.lax.dot_general(
            xb, emb_ref[c * ck:(c + 1) * ck, :], (((1,), (1,)), ((), ())),
            preferred_element_type=jnp.float32,
            precision=jax.lax.Precision.DEFAULT)
        dist = (z2 + e2_ref[0:1, c * ck:(c + 1) * ck]) - 2.0 * s
        cm = dist[:, 0:lanes]
        cidx = base_iota + c * ck
        for i in range(1, ck // lanes):
            d_i = dist[:, i * lanes:(i + 1) * lanes]
            better = d_i < cm
            cm = jnp.where(better, d_i, cm)
            cidx = jnp.where(better, base_iota + (c * ck + i * lanes), cidx)
        if gmin is None:
            gmin, gidx = cm, cidx
        else:
            better = cm < gmin
            gmin = jnp.where(better, cm, gmin)
            gidx = jnp.where(better, cidx, gidx)
    m = jnp.min(gmin, axis=1, keepdims=True)
    idx = jnp.min(jnp.where(gmin == m, gidx, k), axis=1)
    idx_ref[0, 0, :] = idx.astype(jnp.int32)


def _nearest_indices(x, emb):
    n, d = x.shape
    k = emb.shape[0]
    tb = 256
    nb = n // tb
    idx3 = pl.pallas_call(
        _argmin_body,
        grid=(nb,),
        in_specs=[
            pl.BlockSpec((tb, d), lambda i: (i, 0)),
            pl.BlockSpec((k, d), lambda i: (0, 0)),
        ],
        out_specs=pl.BlockSpec((1, 1, tb), lambda i: (i, 0, 0)),
        out_shape=jax.ShapeDtypeStruct((nb, 1, tb), jnp.int32),
        scratch_shapes=[pltpu.VMEM((1, k), jnp.float32)],
    )(x, emb)
    return idx3.reshape(n)


def _sc_gather(emb, idx):
    n = idx.shape[0]
    d = emb.shape[1]
    window = 128
    mesh = plsc.VectorSubcoreMesh(core_axis_name="core",
                                  subcore_axis_name="subcore")
    idx2 = idx.reshape(1, n)

    @pl.kernel(out_type=jax.ShapeDtypeStruct((n, d), emb.dtype), mesh=mesh)
    def gather_kernel(emb_hbm, i_hbm, o_hbm):
        def body(i_vmem, o_vmem):
            pltpu.sync_copy(emb_hbm.at[i_vmem.at[0]], o_vmem)

        pltpu.emit_pipeline(
            body,
            grid=(n // window,),
            in_specs=[pl.BlockSpec((1, window), index_map=lambda i: (0, i))],
            out_specs=[pl.BlockSpec((window, d), index_map=lambda i: (i, 0))],
            core_axis_name=("core", "subcore"),
            dimension_semantics=(pltpu.PARALLEL,),
        )(i_hbm, o_hbm)

    return gather_kernel(emb, idx2)


def _expand_stores(wt_ref, m_ref, cin, cout, hin):
    hout = 2 * hin
    m_ref[...] = jnp.zeros(m_ref.shape, m_ref.dtype)
    for kh in range(4):
        for kw in range(4):
            w = wt_ref[(kh * 4 + kw) * cin:(kh * 4 + kw + 1) * cin, :]
            for ih in range(hin):
                oh = 2 * ih - 1 + kh
                if not 0 <= oh < hout:
                    continue
                for iw in range(hin):
                    ow = 2 * iw - 1 + kw
                    if not 0 <= ow < hout:
                        continue
                    r = (ih * hin + iw) * cin
                    c = (oh * hout + ow) * cout
                    m_ref[r:r + cin, c:c + cout] = w


def _placement_q():
    hin, hout, cout = 8, 16, 3
    q = np.zeros((hin * hin, cout * 16, cout * hout * hout), np.float32)
    for ih in range(hin):
        for iw in range(hin):
            s = ih * hin + iw
            for kh in range(4):
                for kw in range(4):
                    oh, ow = 2 * ih - 1 + kh, 2 * iw - 1 + kw
                    if 0 <= oh < hout and 0 <= ow < hout:
                        k = kh * 4 + kw
                        p = oh * hout + ow
                        for o in range(cout):
                            q[s, o * 16 + k, o * hout * hout + p] = 1.0
    return jnp.asarray(q.astype(jnp.bfloat16))


def _decoder_body(zq_ref, wt1_ref, wt2_ref, wt3_ref, w4_ref, q_ref,
                  b1_ref, b2_ref, b3_ref, b4_ref, out_ref,
                  m1_s, m2_s, m3_s, m4_s, b1_s, b2_s, b3_s, b4_s):
    f32 = jnp.float32
    bf16 = jnp.bfloat16

    @pl.when(pl.program_id(0) == 0)
    def _():
        for p in range(4):
            m1_s[:, p * 256:(p + 1) * 256] = wt1_ref[p * 256:(p + 1) * 256, :]
            b1_s[:, p * 256:(p + 1) * 256] = b1_ref[...]
        _expand_stores(wt2_ref, m2_s, 256, 128, 2)
        _expand_stores(wt3_ref, m3_s, 128, 64, 4)
        for p in range(16):
            b2_s[:, p * 128:(p + 1) * 128] = b2_ref[...]
        for p in range(64):
            b3_s[:, p * 64:(p + 1) * 64] = b3_ref[...]
        w4 = w4_ref[...]
        for s in range(64):
            m4_s[s * 64:(s + 1) * 64, :] = jnp.dot(
                w4, q_ref[s], preferred_element_type=f32).astype(bf16)
        for o in range(3):
            b4_s[:, o * 256:(o + 1) * 256] = jnp.broadcast_to(
                b4_ref[0:1, o:o + 1], (1, 256))

    h = zq_ref[...].astype(bf16)
    h = jnp.dot(h, m1_s[...], preferred_element_type=f32) + b1_s[...]
    h = jnp.maximum(h, 0.0).astype(bf16)
    h = jnp.dot(h, m2_s[...], preferred_element_type=f32) + b2_s[...]
    h = jnp.maximum(h, 0.0).astype(bf16)
    h = jnp.dot(h, m3_s[...], preferred_element_type=f32) + b3_s[...]
    h = jnp.maximum(h, 0.0).astype(bf16)
    h = jnp.dot(h, m4_s[...], preferred_element_type=f32) + b4_s[...]
    out_ref[...] = jax.nn.sigmoid(h)


def _decoder(zq, W1, b1, W2, b2, W3, b3, W4, b4):
    n = zq.shape[0]
    bf16 = jnp.bfloat16
    wt1 = jnp.transpose(W1.astype(bf16)[:, :, 1:3, 1:3],
                        (2, 3, 0, 1)).reshape(4 * 256, 256)
    wt2 = jnp.transpose(W2.astype(bf16), (2, 3, 0, 1)).reshape(16 * 256, 128)
    wt3 = jnp.transpose(W3.astype(bf16), (2, 3, 0, 1)).reshape(16 * 128, 64)
    w4r = W4.astype(bf16).reshape(64, 48)
    q = _placement_q()
    b1r = b1.reshape(1, -1)
    b2r = b2.reshape(1, -1)
    b3r = b3.reshape(1, -1)
    b4r = b4.reshape(1, -1)

    tb = 256
    nb = n // tb
    full = lambda a: pl.BlockSpec(a.shape, lambda i: tuple(0 for _ in a.shape))
    out = pl.pallas_call(
        _decoder_body,
        grid=(nb,),
        in_specs=[pl.BlockSpec((tb, zq.shape[1]), lambda i: (i, 0)),
                  full(wt1), full(wt2), full(wt3), full(w4r), full(q),
                  full(b1r), full(b2r), full(b3r), full(b4r)],
        out_specs=pl.BlockSpec((tb, 768), lambda i: (i, 0)),
        out_shape=jax.ShapeDtypeStruct((n, 768), jnp.float32),
        scratch_shapes=[pltpu.VMEM((256, 1024), bf16),
                        pltpu.VMEM((1024, 2048), bf16),
                        pltpu.VMEM((2048, 4096), bf16),
                        pltpu.VMEM((4096, 768), bf16),
                        pltpu.VMEM((1, 1024), jnp.float32),
                        pltpu.VMEM((1, 2048), jnp.float32),
                        pltpu.VMEM((1, 4096), jnp.float32),
                        pltpu.VMEM((1, 768), jnp.float32)],
    )(zq, wt1, wt2, wt3, w4r, q, b1r, b2r, b3r, b4r)
    return out.reshape(n, W4.shape[1], 16, 16)


def kernel(x, emb, W1, b1, W2, b2, W3, b3, W4, b4):
    n, d = x.shape
    idx = _nearest_indices(x, emb)
    zq = _sc_gather(emb, idx)
    x_recon = _decoder(zq, W1, b1, W2, b2, W3, b3, W4, b4)
    z = x.reshape(n, d, 1, 1)
    return (x_recon, z, zq.reshape(n, d, 1, 1), idx)


# speedup vs baseline: 1.2153x; 1.0257x over previous
import functools

import numpy as np

import jax
import jax.numpy as jnp
from jax.experimental import pallas as pl
from jax.experimental.pallas import tpu as pltpu
from jax.experimental.pallas import tpu_sc as plsc


def _argmin_body(x_ref, emb_ref, idx_ref, e2_ref):
    @pl.when(pl.program_id(0) == 0)
    def _():
        e = emb_ref[...]
        e2_ref[...] = jnp.sum(e * e, axis=1)[None, :]

    xb = x_ref[...]
    tb = xb.shape[0]
    k = emb_ref.shape[0]
    z2 = jnp.sum(xb * xb, axis=1, keepdims=True)

    ck = 1024
    lanes = 128
    base_iota = jax.lax.broadcasted_iota(jnp.int32, (tb, lanes), 1)
    gmin = gidx = None
    for c in range(k // ck):
        s = jax.lax.dot_general(
            xb, emb_ref[c * ck:(c + 1) * ck, :], (((1,), (1,)), ((), ())),
            preferred_element_type=jnp.float32,
            precision=jax.lax.Precision.DEFAULT)
        dist = (z2 + e2_ref[0:1, c * ck:(c + 1) * ck]) - 2.0 * s
        cm = dist[:, 0:lanes]
        cidx = base_iota + c * ck
        for i in range(1, ck // lanes):
            d_i = dist[:, i * lanes:(i + 1) * lanes]
            better = d_i < cm
            cm = jnp.where(better, d_i, cm)
            cidx = jnp.where(better, base_iota + (c * ck + i * lanes), cidx)
        if gmin is None:
            gmin, gidx = cm, cidx
        else:
            better = cm < gmin
            gmin = jnp.where(better, cm, gmin)
            gidx = jnp.where(better, cidx, gidx)
    m = jnp.min(gmin, axis=1, keepdims=True)
    idx = jnp.min(jnp.where(gmin == m, gidx, k), axis=1)
    idx_ref[0, 0, :] = idx.astype(jnp.int32)


def _nearest_indices(x, emb):
    n, d = x.shape
    k = emb.shape[0]
    tb = 512
    nb = n // tb
    idx3 = pl.pallas_call(
        _argmin_body,
        grid=(nb,),
        in_specs=[
            pl.BlockSpec((tb, d), lambda i: (i, 0)),
            pl.BlockSpec((k, d), lambda i: (0, 0)),
        ],
        out_specs=pl.BlockSpec((1, 1, tb), lambda i: (i, 0, 0)),
        out_shape=jax.ShapeDtypeStruct((nb, 1, tb), jnp.int32),
        scratch_shapes=[pltpu.VMEM((1, k), jnp.float32)],
    )(x, emb)
    return idx3.reshape(n)


def _sc_gather(emb, idx):
    n = idx.shape[0]
    d = emb.shape[1]
    window = 128
    mesh = plsc.VectorSubcoreMesh(core_axis_name="core",
                                  subcore_axis_name="subcore")
    idx2 = idx.reshape(1, n)

    @pl.kernel(out_type=jax.ShapeDtypeStruct((n, d), emb.dtype), mesh=mesh)
    def gather_kernel(emb_hbm, i_hbm, o_hbm):
        def body(i_vmem, o_vmem):
            pltpu.sync_copy(emb_hbm.at[i_vmem.at[0]], o_vmem)

        pltpu.emit_pipeline(
            body,
            grid=(n // window,),
            in_specs=[pl.BlockSpec((1, window), index_map=lambda i: (0, i))],
            out_specs=[pl.BlockSpec((window, d), index_map=lambda i: (i, 0))],
            core_axis_name=("core", "subcore"),
            dimension_semantics=(pltpu.PARALLEL,),
        )(i_hbm, o_hbm)

    return gather_kernel(emb, idx2)


def _expand_stores(wt_ref, m_ref, cin, cout, hin):
    hout = 2 * hin
    m_ref[...] = jnp.zeros(m_ref.shape, m_ref.dtype)
    for kh in range(4):
        for kw in range(4):
            w = wt_ref[(kh * 4 + kw) * cin:(kh * 4 + kw + 1) * cin, :]
            for ih in range(hin):
                oh = 2 * ih - 1 + kh
                if not 0 <= oh < hout:
                    continue
                for iw in range(hin):
                    ow = 2 * iw - 1 + kw
                    if not 0 <= ow < hout:
                        continue
                    r = (ih * hin + iw) * cin
                    c = (oh * hout + ow) * cout
                    m_ref[r:r + cin, c:c + cout] = w


def _placement_q():
    hin, hout, cout = 8, 16, 3
    q = np.zeros((hin * hin, cout * 16, cout * hout * hout), np.float32)
    for ih in range(hin):
        for iw in range(hin):
            s = ih * hin + iw
            for kh in range(4):
                for kw in range(4):
                    oh, ow = 2 * ih - 1 + kh, 2 * iw - 1 + kw
                    if 0 <= oh < hout and 0 <= ow < hout:
                        k = kh * 4 + kw
                        p = oh * hout + ow
                        for o in range(cout):
                            q[s, o * 16 + k, o * hout * hout + p] = 1.0
    return jnp.asarray(q.astype(jnp.bfloat16))


def _decoder_body(zq_ref, wt1_ref, wt2_ref, wt3_ref, w4_ref, q_ref,
                  b1_ref, b2_ref, b3_ref, b4_ref, out_ref,
                  m1_s, m2_s, m3_s, m4_s, b1_s, b2_s, b3_s, b4_s):
    f32 = jnp.float32
    bf16 = jnp.bfloat16

    @pl.when(pl.program_id(0) == 0)
    def _():
        for p in range(4):
            m1_s[:, p * 256:(p + 1) * 256] = wt1_ref[p * 256:(p + 1) * 256, :]
            b1_s[:, p * 256:(p + 1) * 256] = b1_ref[...]
        _expand_stores(wt2_ref, m2_s, 256, 128, 2)
        _expand_stores(wt3_ref, m3_s, 128, 64, 4)
        for p in range(16):
            b2_s[:, p * 128:(p + 1) * 128] = b2_ref[...]
        for p in range(64):
            b3_s[:, p * 64:(p + 1) * 64] = b3_ref[...]
        w4 = w4_ref[...]
        for s in range(64):
            m4_s[s * 64:(s + 1) * 64, :] = jnp.dot(
                w4, q_ref[s], preferred_element_type=f32).astype(bf16)
        for o in range(3):
            b4_s[:, o * 256:(o + 1) * 256] = jnp.broadcast_to(
                b4_ref[0:1, o:o + 1], (1, 256))

    h = zq_ref[...].astype(bf16)
    h = jnp.dot(h, m1_s[...], preferred_element_type=f32) + b1_s[...]
    h = jnp.maximum(h, 0.0).astype(bf16)
    h = jnp.dot(h, m2_s[...], preferred_element_type=f32) + b2_s[...]
    h = jnp.maximum(h, 0.0).astype(bf16)
    h = jnp.dot(h, m3_s[...], preferred_element_type=f32) + b3_s[...]
    h = jnp.maximum(h, 0.0).astype(bf16)
    h = jnp.dot(h, m4_s[...], preferred_element_type=f32) + b4_s[...]
    out_ref[...] = jax.nn.sigmoid(h)


def _decoder(zq, W1, b1, W2, b2, W3, b3, W4, b4):
    n = zq.shape[0]
    bf16 = jnp.bfloat16
    wt1 = jnp.transpose(W1.astype(bf16)[:, :, 1:3, 1:3],
                        (2, 3, 0, 1)).reshape(4 * 256, 256)
    wt2 = jnp.transpose(W2.astype(bf16), (2, 3, 0, 1)).reshape(16 * 256, 128)
    wt3 = jnp.transpose(W3.astype(bf16), (2, 3, 0, 1)).reshape(16 * 128, 64)
    w4r = W4.astype(bf16).reshape(64, 48)
    q = _placement_q()
    b1r = b1.reshape(1, -1)
    b2r = b2.reshape(1, -1)
    b3r = b3.reshape(1, -1)
    b4r = b4.reshape(1, -1)

    tb = 512
    nb = n // tb
    full = lambda a: pl.BlockSpec(a.shape, lambda i: tuple(0 for _ in a.shape))
    out = pl.pallas_call(
        _decoder_body,
        grid=(nb,),
        in_specs=[pl.BlockSpec((tb, zq.shape[1]), lambda i: (i, 0)),
                  full(wt1), full(wt2), full(wt3), full(w4r), full(q),
                  full(b1r), full(b2r), full(b3r), full(b4r)],
        out_specs=pl.BlockSpec((tb, 768), lambda i: (i, 0)),
        out_shape=jax.ShapeDtypeStruct((n, 768), jnp.float32),
        scratch_shapes=[pltpu.VMEM((256, 1024), bf16),
                        pltpu.VMEM((1024, 2048), bf16),
                        pltpu.VMEM((2048, 4096), bf16),
                        pltpu.VMEM((4096, 768), bf16),
                        pltpu.VMEM((1, 1024), jnp.float32),
                        pltpu.VMEM((1, 2048), jnp.float32),
                        pltpu.VMEM((1, 4096), jnp.float32),
                        pltpu.VMEM((1, 768), jnp.float32)],
    )(zq, wt1, wt2, wt3, w4r, q, b1r, b2r, b3r, b4r)
    return out.reshape(n, W4.shape[1], 16, 16)


def kernel(x, emb, W1, b1, W2, b2, W3, b3, W4, b4):
    n, d = x.shape
    idx = _nearest_indices(x, emb)
    zq = _sc_gather(emb, idx)
    x_recon = _decoder(zq, W1, b1, W2, b2, W3, b3, W4, b4)
    z = x.reshape(n, d, 1, 1)
    return (x_recon, z, zq.reshape(n, d, 1, 1), idx)


# speedup vs baseline: 1.2164x; 1.0010x over previous
import functools

import numpy as np

import jax
import jax.numpy as jnp
from jax.experimental import pallas as pl
from jax.experimental.pallas import tpu as pltpu
from jax.experimental.pallas import tpu_sc as plsc


def _argmin_body(x_ref, emb_ref, idx_ref, e2_ref):
    @pl.when(pl.program_id(0) == 0)
    def _():
        e = emb_ref[...]
        e2_ref[...] = jnp.sum(e * e, axis=1)[None, :]

    xb = x_ref[...]
    tb = xb.shape[0]
    k = emb_ref.shape[0]
    z2 = jnp.sum(xb * xb, axis=1, keepdims=True)

    ck = 2048
    lanes = 128
    base_iota = jax.lax.broadcasted_iota(jnp.int32, (tb, lanes), 1)
    gmin = gidx = None
    for c in range(k // ck):
        s = jax.lax.dot_general(
            xb, emb_ref[c * ck:(c + 1) * ck, :], (((1,), (1,)), ((), ())),
            preferred_element_type=jnp.float32,
            precision=jax.lax.Precision.DEFAULT)
        dist = (z2 + e2_ref[0:1, c * ck:(c + 1) * ck]) - 2.0 * s
        cm = dist[:, 0:lanes]
        cidx = base_iota + c * ck
        for i in range(1, ck // lanes):
            d_i = dist[:, i * lanes:(i + 1) * lanes]
            better = d_i < cm
            cm = jnp.where(better, d_i, cm)
            cidx = jnp.where(better, base_iota + (c * ck + i * lanes), cidx)
        if gmin is None:
            gmin, gidx = cm, cidx
        else:
            better = cm < gmin
            gmin = jnp.where(better, cm, gmin)
            gidx = jnp.where(better, cidx, gidx)
    m = jnp.min(gmin, axis=1, keepdims=True)
    idx = jnp.min(jnp.where(gmin == m, gidx, k), axis=1)
    idx_ref[0, 0, :] = idx.astype(jnp.int32)


def _nearest_indices(x, emb):
    n, d = x.shape
    k = emb.shape[0]
    tb = 512
    nb = n // tb
    idx3 = pl.pallas_call(
        _argmin_body,
        grid=(nb,),
        in_specs=[
            pl.BlockSpec((tb, d), lambda i: (i, 0)),
            pl.BlockSpec((k, d), lambda i: (0, 0)),
        ],
        out_specs=pl.BlockSpec((1, 1, tb), lambda i: (i, 0, 0)),
        out_shape=jax.ShapeDtypeStruct((nb, 1, tb), jnp.int32),
        scratch_shapes=[pltpu.VMEM((1, k), jnp.float32)],
    )(x, emb)
    return idx3.reshape(n)


def _sc_gather(emb, idx):
    n = idx.shape[0]
    d = emb.shape[1]
    window = 128
    mesh = plsc.VectorSubcoreMesh(core_axis_name="core",
                                  subcore_axis_name="subcore")
    idx2 = idx.reshape(1, n)

    @pl.kernel(out_type=jax.ShapeDtypeStruct((n, d), emb.dtype), mesh=mesh)
    def gather_kernel(emb_hbm, i_hbm, o_hbm):
        def body(i_vmem, o_vmem):
            pltpu.sync_copy(emb_hbm.at[i_vmem.at[0]], o_vmem)

        pltpu.emit_pipeline(
            body,
            grid=(n // window,),
            in_specs=[pl.BlockSpec((1, window), index_map=lambda i: (0, i))],
            out_specs=[pl.BlockSpec((window, d), index_map=lambda i: (i, 0))],
            core_axis_name=("core", "subcore"),
            dimension_semantics=(pltpu.PARALLEL,),
        )(i_hbm, o_hbm)

    return gather_kernel(emb, idx2)


def _expand_stores(wt_ref, m_ref, cin, cout, hin):
    hout = 2 * hin
    m_ref[...] = jnp.zeros(m_ref.shape, m_ref.dtype)
    for kh in range(4):
        for kw in range(4):
            w = wt_ref[(kh * 4 + kw) * cin:(kh * 4 + kw + 1) * cin, :]
            for ih in range(hin):
                oh = 2 * ih - 1 + kh
                if not 0 <= oh < hout:
                    continue
                for iw in range(hin):
                    ow = 2 * iw - 1 + kw
                    if not 0 <= ow < hout:
                        continue
                    r = (ih * hin + iw) * cin
                    c = (oh * hout + ow) * cout
                    m_ref[r:r + cin, c:c + cout] = w


def _placement_q():
    hin, hout, cout = 8, 16, 3
    q = np.zeros((hin * hin, cout * 16, cout * hout * hout), np.float32)
    for ih in range(hin):
        for iw in range(hin):
            s = ih * hin + iw
            for kh in range(4):
                for kw in range(4):
                    oh, ow = 2 * ih - 1 + kh, 2 * iw - 1 + kw
                    if 0 <= oh < hout and 0 <= ow < hout:
                        k = kh * 4 + kw
                        p = oh * hout + ow
                        for o in range(cout):
                            q[s, o * 16 + k, o * hout * hout + p] = 1.0
    return jnp.asarray(q.astype(jnp.bfloat16))


def _decoder_body(zq_ref, wt1_ref, wt2_ref, wt3_ref, w4_ref, q_ref,
                  b1_ref, b2_ref, b3_ref, b4_ref, out_ref,
                  m1_s, m2_s, m3_s, m4_s, b1_s, b2_s, b3_s, b4_s):
    f32 = jnp.float32
    bf16 = jnp.bfloat16

    @pl.when(pl.program_id(0) == 0)
    def _():
        for p in range(4):
            m1_s[:, p * 256:(p + 1) * 256] = wt1_ref[p * 256:(p + 1) * 256, :]
            b1_s[:, p * 256:(p + 1) * 256] = b1_ref[...]
        _expand_stores(wt2_ref, m2_s, 256, 128, 2)
        _expand_stores(wt3_ref, m3_s, 128, 64, 4)
        for p in range(16):
            b2_s[:, p * 128:(p + 1) * 128] = b2_ref[...]
        for p in range(64):
            b3_s[:, p * 64:(p + 1) * 64] = b3_ref[...]
        w4 = w4_ref[...]
        for s in range(64):
            m4_s[s * 64:(s + 1) * 64, :] = jnp.dot(
                w4, q_ref[s], preferred_element_type=f32).astype(bf16)
        for o in range(3):
            b4_s[:, o * 256:(o + 1) * 256] = jnp.broadcast_to(
                b4_ref[0:1, o:o + 1], (1, 256))

    h = zq_ref[...].astype(bf16)
    h = jnp.dot(h, m1_s[...], preferred_element_type=f32) + b1_s[...]
    h = jnp.maximum(h, 0.0).astype(bf16)
    h = jnp.dot(h, m2_s[...], preferred_element_type=f32) + b2_s[...]
    h = jnp.maximum(h, 0.0).astype(bf16)
    h = jnp.dot(h, m3_s[...], preferred_element_type=f32) + b3_s[...]
    h = jnp.maximum(h, 0.0).astype(bf16)
    h = jnp.dot(h, m4_s[...], preferred_element_type=f32) + b4_s[...]
    out_ref[...] = jax.nn.sigmoid(h)


def _decoder(zq, W1, b1, W2, b2, W3, b3, W4, b4):
    n = zq.shape[0]
    bf16 = jnp.bfloat16
    wt1 = jnp.transpose(W1.astype(bf16)[:, :, 1:3, 1:3],
                        (2, 3, 0, 1)).reshape(4 * 256, 256)
    wt2 = jnp.transpose(W2.astype(bf16), (2, 3, 0, 1)).reshape(16 * 256, 128)
    wt3 = jnp.transpose(W3.astype(bf16), (2, 3, 0, 1)).reshape(16 * 128, 64)
    w4r = W4.astype(bf16).reshape(64, 48)
    q = _placement_q()
    b1r = b1.reshape(1, -1)
    b2r = b2.reshape(1, -1)
    b3r = b3.reshape(1, -1)
    b4r = b4.reshape(1, -1)

    tb = 512
    nb = n // tb
    full = lambda a: pl.BlockSpec(a.shape, lambda i: tuple(0 for _ in a.shape))
    out = pl.pallas_call(
        _decoder_body,
        grid=(nb,),
        in_specs=[pl.BlockSpec((tb, zq.shape[1]), lambda i: (i, 0)),
                  full(wt1), full(wt2), full(wt3), full(w4r), full(q),
                  full(b1r), full(b2r), full(b3r), full(b4r)],
        out_specs=pl.BlockSpec((tb, 768), lambda i: (i, 0)),
        out_shape=jax.ShapeDtypeStruct((n, 768), jnp.float32),
        scratch_shapes=[pltpu.VMEM((256, 1024), bf16),
                        pltpu.VMEM((1024, 2048), bf16),
                        pltpu.VMEM((2048, 4096), bf16),
                        pltpu.VMEM((4096, 768), bf16),
                        pltpu.VMEM((1, 1024), jnp.float32),
                        pltpu.VMEM((1, 2048), jnp.float32),
                        pltpu.VMEM((1, 4096), jnp.float32),
                        pltpu.VMEM((1, 768), jnp.float32)],
    )(zq, wt1, wt2, wt3, w4r, q, b1r, b2r, b3r, b4r)
    return out.reshape(n, W4.shape[1], 16, 16)


def kernel(x, emb, W1, b1, W2, b2, W3, b3, W4, b4):
    n, d = x.shape
    idx = _nearest_indices(x, emb)
    zq = _sc_gather(emb, idx)
    x_recon = _decoder(zq, W1, b1, W2, b2, W3, b3, W4, b4)
    z = x.reshape(n, d, 1, 1)
    return (x_recon, z, zq.reshape(n, d, 1, 1), idx)
